# Initial kernel scaffold; baseline (speedup 1.0000x reference)
import jax
import jax.numpy as jnp
from jax.experimental import pallas as pl


def kernel(edge_feat, node_feat, graph_attr, W_e, b_e, W_n, b_n, W_a, b_a, edge_index):
    raise NotImplementedError("write your pallas kernel here")


# speedup vs baseline: 4.4405x; 4.4405x over previous
import functools

import jax
import jax.numpy as jnp
from jax import lax
from jax.experimental import pallas as pl
from jax.experimental.pallas import tpu as pltpu
from jax.experimental.pallas import tpu_sc as plsc

N = 10000
E = 320000
DV = 128
DE = 16
DU = 32
H = 32

NC = 2
NS = 16
NW = NC * NS
EW = E // NW
C = 400
K = EW // C
S = 100
SUB = C // S
NRF = 1000
NFT = N // NRF

_LOG1P = (0.04106444225260315, -0.15602827499078686, 0.30467224693119505,
          -0.4963682486301464, 0.9998879230599648)


def _softplus_vec(z):
    t = jnp.exp(-jnp.abs(z))
    q = jnp.float32(_LOG1P[0])
    for c in _LOG1P[1:]:
        q = q * t + jnp.float32(c)
    return jnp.maximum(z, jnp.float32(0.0)) + t * q


def _proj_body(nf_ref, wcat_ref, p1_ref, p2_ref, pn_ref):
    p = jnp.dot(nf_ref[...], wcat_ref[...], preferred_element_type=jnp.float32)
    p1_ref[...] = p[:, 0:H]
    p2_ref[...] = p[:, H:2 * H]
    pn_ref[...] = p[:, 2 * H:3 * H]


def _ep_body(ef_ref, wee_ref, u_ref, weu_ref, be_ref, ep_ref):
    ce = jnp.dot(u_ref[...], weu_ref[...], preferred_element_type=jnp.float32) + be_ref[...]
    ep_ref[...] = jnp.dot(ef_ref[...], wee_ref[...], preferred_element_type=jnp.float32) + ce


def _sc_edge_body(src_hbm, dst_hbm, p1_hbm, p2_hbm, ep_hbm, ones_hbm,
                  z32_hbm, z8_hbm,
                  enew_hbm, sums_hbm, cnt_hbm,
                  src_v, dst_v, buf1, buf2, bufp, ones_v,
                  sums_sp, cnt_sp, sem1, sem2, sem3):
    cid = lax.axis_index("c")
    sid = lax.axis_index("s")
    wid = sid * NC + cid

    @pl.when(sid < NFT)
    def _():
        pltpu.sync_copy(z32_hbm, sums_sp.at[pl.ds(sid * NRF, NRF)])
        pltpu.sync_copy(z8_hbm, cnt_sp.at[pl.ds(sid * NRF, NRF)])
    pltpu.sync_copy(ones_hbm, ones_v)
    plsc.subcore_barrier()

    def chunk_body(k, carry):
        base = wid * EW + k * C
        rb = wid * (EW // S) + k * SUB
        pltpu.sync_copy(src_hbm.at[pl.ds(rb, SUB)], src_v)
        pltpu.sync_copy(dst_hbm.at[pl.ds(rb, SUB)], dst_v)
        gathers = []
        for j in range(SUB):
            gathers.append(pltpu.async_copy(
                p1_hbm.at[src_v.at[j]], buf1.at[pl.ds(j * S, S)], sem1))
            gathers.append(pltpu.async_copy(
                p2_hbm.at[dst_v.at[j]], buf2.at[pl.ds(j * S, S)], sem2))
        dp = pltpu.async_copy(ep_hbm.at[pl.ds(base, C)], bufp, sem3)
        for g in gathers:
            g.wait()
        dp.wait()

        def row_body(r, c2):
            for h in (0, H // 2):
                z = (buf1[r, pl.ds(h, 16)] + buf2[r, pl.ds(h, 16)]
                     + bufp[r, pl.ds(h, 16)])
                buf1[r, pl.ds(h, 16)] = _softplus_vec(z)
            return c2
        lax.fori_loop(0, C, row_body, 0)

        pltpu.sync_copy(buf1, enew_hbm.at[pl.ds(base, C)])
        for j in range(SUB):
            pltpu.sync_copy(buf1.at[pl.ds(j * S, S)],
                            sums_sp.at[dst_v.at[j]], add=True)
            pltpu.sync_copy(ones_v, cnt_sp.at[dst_v.at[j]], add=True)
        return carry

    lax.fori_loop(0, K, chunk_body, 0)
    plsc.subcore_barrier()

    @pl.when(sid < NFT)
    def _():
        pltpu.sync_copy(sums_sp.at[pl.ds(sid * NRF, NRF)],
                        sums_hbm.at[cid, pl.ds(sid * NRF, NRF)])
        pltpu.sync_copy(cnt_sp.at[pl.ds(sid * NRF, NRF)],
                        cnt_hbm.at[cid, pl.ds(sid * NRF, NRF)])


_sc_edge = functools.partial(
    pl.kernel,
    out_type=(jax.ShapeDtypeStruct((E, H), jnp.float32),
              jax.ShapeDtypeStruct((NC, N, H), jnp.float32),
              jax.ShapeDtypeStruct((NC, N, 8), jnp.float32)),
    mesh=plsc.VectorSubcoreMesh(core_axis_name="c", subcore_axis_name="s"),
    compiler_params=pltpu.CompilerParams(use_tc_tiling_on_sc=False),
    scratch_types=(
        pltpu.VMEM((SUB, S), jnp.int32),
        pltpu.VMEM((SUB, S), jnp.int32),
        pltpu.VMEM((C, H), jnp.float32),
        pltpu.VMEM((C, H), jnp.float32),
        pltpu.VMEM((C, H), jnp.float32),
        pltpu.VMEM((S, 8), jnp.float32),
        pltpu.VMEM_SHARED((N, H), jnp.float32),
        pltpu.VMEM_SHARED((N, 8), jnp.float32),
        pltpu.SemaphoreType.DMA,
        pltpu.SemaphoreType.DMA,
        pltpu.SemaphoreType.DMA,
    ),
)(_sc_edge_body)


def _final_body(sums_ref, cnt_ref, pn_ref, u_ref, wn2_ref, wnu_ref, bn_ref,
                wa1_ref, wa2_ref, wa3_ref, ba_ref, v_ref, u_new_ref):
    def sp(x):
        return jnp.maximum(x, 0.0) + jnp.log(1.0 + jnp.exp(-jnp.abs(x)))

    s = sums_ref[0] + sums_ref[1]
    c8 = cnt_ref[0] + cnt_ref[1]
    cnt = c8[:, 0:1]
    ve = s / jnp.maximum(cnt, 1.0)
    u = u_ref[...]
    cn = jnp.dot(u, wnu_ref[...], preferred_element_type=jnp.float32) + bn_ref[...]
    v_new = sp(pn_ref[...]
               + jnp.dot(ve, wn2_ref[...], preferred_element_type=jnp.float32)
               + cn)
    v_ref[...] = v_new
    ue = jnp.sum(s, axis=0, keepdims=True) * (1.0 / E)
    uv = jnp.sum(v_new, axis=0, keepdims=True) * (1.0 / N)
    u_new_ref[...] = sp(jnp.dot(u, wa1_ref[...], preferred_element_type=jnp.float32)
                        + jnp.dot(ue, wa2_ref[...], preferred_element_type=jnp.float32)
                        + jnp.dot(uv, wa3_ref[...], preferred_element_type=jnp.float32)
                        + ba_ref[...])


def kernel(edge_feat, node_feat, graph_attr, W_e, b_e, W_n, b_n, W_a, b_a,
           edge_index):
    f32 = jnp.float32
    wcat = jnp.concatenate(
        [W_e[0:DV], W_e[DV:2 * DV], W_n[0:DV]], axis=1)
    p1, p2, pn = pl.pallas_call(
        _proj_body,
        out_shape=(jax.ShapeDtypeStruct((N, H), f32),
                   jax.ShapeDtypeStruct((N, H), f32),
                   jax.ShapeDtypeStruct((N, H), f32)),
    )(node_feat, wcat)

    ep = pl.pallas_call(
        _ep_body,
        grid=(E // 8000,),
        in_specs=[
            pl.BlockSpec((8000, DE), lambda i: (i, 0)),
            pl.BlockSpec((DE, H), lambda i: (0, 0)),
            pl.BlockSpec((1, DU), lambda i: (0, 0)),
            pl.BlockSpec((DU, H), lambda i: (0, 0)),
            pl.BlockSpec((1, H), lambda i: (0, 0)),
        ],
        out_specs=pl.BlockSpec((8000, H), lambda i: (i, 0)),
        out_shape=jax.ShapeDtypeStruct((E, H), f32),
    )(edge_feat, W_e[2 * DV:2 * DV + DE], graph_attr,
      W_e[2 * DV + DE:], b_e.reshape(1, H))

    src = edge_index[0].reshape(E // S, S).astype(jnp.int32)
    dst = edge_index[1].reshape(E // S, S).astype(jnp.int32)
    ones = jnp.ones((S, 8), f32)
    z32 = jnp.zeros((NRF, H), f32)
    z8 = jnp.zeros((NRF, 8), f32)

    e_new, sums, cnt = _sc_edge(src, dst, p1, p2, ep, ones, z32, z8)

    v_new, u_new = pl.pallas_call(
        _final_body,
        out_shape=(jax.ShapeDtypeStruct((N, H), f32),
                   jax.ShapeDtypeStruct((1, H), f32)),
    )(sums, cnt, pn, graph_attr,
      W_n[DV:DV + H], W_n[DV + H:], b_n.reshape(1, H),
      W_a[0:DU], W_a[DU:DU + H], W_a[DU + H:], b_a.reshape(1, H))

    return (e_new, v_new, u_new)
